# baseline (speedup 1.0000x reference)
import functools

import jax
import jax.numpy as jnp
from jax import lax
from jax.experimental import pallas as pl
from jax.experimental.pallas import tpu as pltpu

_ROWS = 128
_COLS = 100000
_BLOCK = 2048
_NSTEPS = (_COLS + _BLOCK - 1) // _BLOCK


def _threefry2x32(k1, k2, x0, x1):
    def rotl(x, d):
        return (x << jnp.uint32(d)) | (x >> jnp.uint32(32 - d))

    def rounds(v0, v1, rots):
        for r in rots:
            v0 = v0 + v1
            v1 = rotl(v1, r)
            v1 = v0 ^ v1
        return v0, v1

    rot_a = (13, 15, 26, 6)
    rot_b = (17, 29, 16, 24)
    ks0 = k1
    ks1 = k2
    ks2 = k1 ^ k2 ^ jnp.uint32(0x1BD11BDA)
    v0 = x0 + ks0
    v1 = x1 + ks1
    v0, v1 = rounds(v0, v1, rot_a)
    v0 = v0 + ks1
    v1 = v1 + (ks2 + jnp.uint32(1))
    v0, v1 = rounds(v0, v1, rot_b)
    v0 = v0 + ks2
    v1 = v1 + (ks0 + jnp.uint32(2))
    v0, v1 = rounds(v0, v1, rot_a)
    v0 = v0 + ks0
    v1 = v1 + (ks1 + jnp.uint32(3))
    v0, v1 = rounds(v0, v1, rot_b)
    v0 = v0 + ks1
    v1 = v1 + (ks2 + jnp.uint32(4))
    v0, v1 = rounds(v0, v1, rot_a)
    v0 = v0 + ks2
    v1 = v1 + (ks0 + jnp.uint32(5))
    return v0, v1


def _sampler_kernel(key_ref, logits_ref, out_ref, vmax_ref, vidx_ref):
    step = pl.program_id(0)
    k1 = key_ref[0]
    k2 = key_ref[1]

    col = lax.broadcasted_iota(jnp.int32, (_ROWS, _BLOCK), 1) + step * _BLOCK
    row = lax.broadcasted_iota(jnp.int32, (_ROWS, _BLOCK), 0)
    lo = (row * _COLS + col).astype(jnp.uint32)
    hi = jnp.zeros((_ROWS, _BLOCK), jnp.uint32)

    o0, o1 = _threefry2x32(k1, k2, hi, lo)
    bits = o0 ^ o1

    fb = (bits >> jnp.uint32(9)) | jnp.uint32(0x3F800000)
    u = lax.bitcast_convert_type(fb, jnp.float32) - jnp.float32(1.0)
    u = u * jnp.float32(1.0 - 1e-20) + jnp.float32(1e-20)
    u = jnp.maximum(jnp.float32(1e-20), u)

    gumbel = -jnp.log(-jnp.log(u))
    val = logits_ref[...] + gumbel
    val = jnp.where(col < _COLS, val, -jnp.inf)

    m = jnp.max(val, axis=1, keepdims=True)
    idx = jnp.min(
        jnp.where(val == m, col, jnp.int32(2**31 - 1)), axis=1, keepdims=True
    )

    @pl.when(step == 0)
    def _init():
        vmax_ref[...] = m
        vidx_ref[...] = idx

    @pl.when(step > 0)
    def _update():
        upd = m > vmax_ref[...]
        vmax_ref[...] = jnp.where(upd, m, vmax_ref[...])
        vidx_ref[...] = jnp.where(upd, idx, vidx_ref[...])

    @pl.when(step == _NSTEPS - 1)
    def _finish():
        out_ref[...] = vidx_ref[...]


@jax.jit
def kernel(logits):
    key = jax.random.key_data(jax.random.key(42)).astype(jnp.uint32)
    samples = pl.pallas_call(
        _sampler_kernel,
        grid=(_NSTEPS,),
        in_specs=[
            pl.BlockSpec(memory_space=pltpu.SMEM),
            pl.BlockSpec((_ROWS, _BLOCK), lambda i: (0, i)),
        ],
        out_specs=pl.BlockSpec((_ROWS, 1), lambda i: (0, 0)),
        out_shape=jax.ShapeDtypeStruct((_ROWS, 1), jnp.int32),
        scratch_shapes=[
            pltpu.VMEM((_ROWS, 1), jnp.float32),
            pltpu.VMEM((_ROWS, 1), jnp.int32),
        ],
        compiler_params=pltpu.CompilerParams(
            dimension_semantics=("arbitrary",),
        ),
    )(key, logits)
    return samples.reshape(_ROWS)
